# Initial kernel scaffold; baseline (speedup 1.0000x reference)
import jax
import jax.numpy as jnp
from jax.experimental import pallas as pl


def kernel(x, edge_index, edge_attr, batch, W1_1, b1_1, W2_1, W3_1, b3_1, W1_2, b1_2, W2_2, W3_2, b3_2):
    raise NotImplementedError("write your pallas kernel here")


# speedup vs baseline: 2.5545x; 2.5545x over previous
import functools

import jax
import jax.numpy as jnp
from jax import lax
from jax.experimental import pallas as pl
from jax.experimental.pallas import tpu as pltpu
from jax.experimental.pallas import tpu_sc as plsc

N = 10000
E = 160000
DIM = 256
HD = 128
NC = 2
NS = 16
L = 16
EPT = E // NS
C = 80
NCHUNK = EPT // C
SB = 5
NBATCH = NCHUNK // SB
NROWCH = N // C
RCPT = -(-NROWCH // NS)


_BN = 1000


def _mm3_body(x_ref, w_ref, b_ref, a_ref, bb_ref, c_ref):
    acc = jnp.dot(x_ref[...], w_ref[...],
                  preferred_element_type=jnp.float32,
                  precision=lax.Precision.HIGHEST)
    acc = acc + b_ref[...]
    a_ref[...] = acc[:, 0:DIM]
    bb_ref[...] = acc[:, DIM:2 * DIM]
    c_ref[...] = acc[:, 2 * DIM:3 * DIM]


def _mm3(x, wcat, bcat):
    return pl.pallas_call(
        _mm3_body,
        grid=(N // _BN,),
        in_specs=[
            pl.BlockSpec((_BN, DIM), lambda i: (i, 0)),
            pl.BlockSpec((DIM, 3 * DIM), lambda i: (0, 0)),
            pl.BlockSpec((1, 3 * DIM), lambda i: (0, 0)),
        ],
        out_specs=[
            pl.BlockSpec((_BN, DIM), lambda i: (i, 0)),
            pl.BlockSpec((_BN, DIM), lambda i: (i, 0)),
            pl.BlockSpec((_BN, DIM), lambda i: (i, 0)),
        ],
        out_shape=[jax.ShapeDtypeStruct((N, DIM), jnp.float32)] * 3,
    )(x, wcat, bcat)


def _combine(s_ref, c_ref):
    left = s_ref[0] + c_ref[:, 0:HD]
    right = s_ref[1] + c_ref[:, HD:DIM]
    return jnp.concatenate([left, right], axis=1)


def _comb_mm_body(s_ref, c_ref, w_ref, b_ref, a_ref, bb2_ref, c2_ref):
    h = jnp.maximum(_combine(s_ref, c_ref), 0.0)
    acc = jnp.dot(h, w_ref[...],
                  preferred_element_type=jnp.float32,
                  precision=lax.Precision.HIGHEST)
    acc = acc + b_ref[...]
    a_ref[...] = acc[:, 0:DIM]
    bb2_ref[...] = acc[:, DIM:2 * DIM]
    c2_ref[...] = acc[:, 2 * DIM:3 * DIM]


def _comb_mm(s, c, wcat, bcat):
    return pl.pallas_call(
        _comb_mm_body,
        grid=(N // _BN,),
        in_specs=[
            pl.BlockSpec((NC, _BN, HD), lambda i: (0, i, 0)),
            pl.BlockSpec((_BN, DIM), lambda i: (i, 0)),
            pl.BlockSpec((DIM, 3 * DIM), lambda i: (0, 0)),
            pl.BlockSpec((1, 3 * DIM), lambda i: (0, 0)),
        ],
        out_specs=[
            pl.BlockSpec((_BN, DIM), lambda i: (i, 0)),
            pl.BlockSpec((_BN, DIM), lambda i: (i, 0)),
            pl.BlockSpec((_BN, DIM), lambda i: (i, 0)),
        ],
        out_shape=[jax.ShapeDtypeStruct((N, DIM), jnp.float32)] * 3,
    )(s, c, wcat, bcat)


def _final_body(s_ref, c_ref, out_ref):
    out_ref[...] = _combine(s_ref, c_ref)


def _final(s, c):
    return pl.pallas_call(
        _final_body,
        grid=(N // _BN,),
        in_specs=[
            pl.BlockSpec((NC, _BN, HD), lambda i: (0, i, 0)),
            pl.BlockSpec((_BN, DIM), lambda i: (i, 0)),
        ],
        out_specs=pl.BlockSpec((_BN, DIM), lambda i: (i, 0)),
        out_shape=jax.ShapeDtypeStruct((N, DIM), jnp.float32),
    )(s, c)


_SC_MESH = plsc.VectorSubcoreMesh(
    core_axis_name="c", subcore_axis_name="s", num_cores=NC, num_subcores=NS)


@functools.partial(
    pl.kernel,
    out_type=jax.ShapeDtypeStruct((NC, N, HD), jnp.float32),
    mesh=_SC_MESH,
    scratch_types=[
        pltpu.VMEM((SB * C,), jnp.int32),
        pltpu.VMEM((SB * C,), jnp.int32),
        pltpu.VMEM((SB * C,), jnp.float32),
        pltpu.VMEM((C,), jnp.int32),
        pltpu.VMEM((C,), jnp.int32),
        pltpu.VMEM((C,), jnp.int32),
        pltpu.VMEM((C, HD), jnp.float32),
        pltpu.VMEM((C, HD), jnp.float32),
        pltpu.VMEM_SHARED((N, HD), jnp.float32),
        pltpu.SemaphoreType.DMA,
        pltpu.SemaphoreType.DMA,
    ],
)
def _sc_spmm(a_hbm, b_hbm, src_hbm, dst_hbm, ew_hbm, s_out,
             src_b, dst_b, ew_b, gidx_c, bidx_c, dst_c, rows_a, rows_b,
             s_sh, sem_a, sem_b):
    cid = lax.axis_index("c")
    sid = lax.axis_index("s")
    ebase = sid * EPT

    zero = jnp.zeros((L,), jnp.float32)

    def _zrows_body(e, carry):
        for k in range(HD // L):
            rows_a[e, pl.ds(k * L, L)] = zero
        return carry
    lax.fori_loop(0, C, _zrows_body, 0)

    for j in range(RCPT):
        rch = sid + j * NS

        @pl.when(rch < NROWCH)
        def _zero():
            pltpu.sync_copy(rows_a, s_sh.at[pl.ds(rch * C, C)])
    plsc.subcore_barrier()

    def _batch_body(b, carry):
        boff = ebase + b * SB * C
        pltpu.sync_copy(src_hbm.at[pl.ds(boff, SB * C)], src_b)
        pltpu.sync_copy(dst_hbm.at[pl.ds(boff, SB * C)], dst_b)
        pltpu.sync_copy(ew_hbm.at[pl.ds(boff, SB * C)], ew_b)

        def _chunk_body(j, carry2):
            co = pl.multiple_of(j * C, C)
            for k in range(C // L):
                sv = src_b[pl.ds(co + k * L, L)]
                dv = dst_b[pl.ds(co + k * L, L)]
                gidx_c[pl.ds(k * L, L)] = sv * 2 + cid
                bidx_c[pl.ds(k * L, L)] = dv * 2 + cid
                dst_c[pl.ds(k * L, L)] = dv
            cp_a = pltpu.async_copy(a_hbm.at[gidx_c], rows_a, sem_a)
            cp_b = pltpu.async_copy(b_hbm.at[bidx_c], rows_b, sem_b)
            cp_a.wait()
            cp_b.wait()
            for g in range(C // L):
                wv = ew_b[pl.ds(co + g * L, L)]
                for lane in range(L):
                    w = jnp.full((L,), wv[lane])
                    e = g * L + lane
                    for k in range(HD // L):
                        rows_a[e, pl.ds(k * L, L)] = (
                            rows_a[e, pl.ds(k * L, L)]
                            - rows_b[e, pl.ds(k * L, L)]) * w
            pltpu.sync_copy(rows_a, s_sh.at[dst_c], add=True)
            return carry2
        lax.fori_loop(0, SB, _chunk_body, 0)
        return carry
    lax.fori_loop(0, NBATCH, _batch_body, 0)
    plsc.subcore_barrier()

    for j in range(RCPT):
        rch = sid + j * NS

        @pl.when(rch < NROWCH)
        def _wb():
            pltpu.sync_copy(s_sh.at[pl.ds(rch * C, C)], rows_a)
            pltpu.sync_copy(rows_a, s_out.at[cid, pl.ds(rch * C, C)])


def kernel(x, edge_index, edge_attr, batch,
           W1_1, b1_1, W2_1, W3_1, b3_1,
           W1_2, b1_2, W2_2, W3_2, b3_2):
    del batch
    src = edge_index[0]
    dst = edge_index[1]
    ew = edge_attr.reshape(E)

    wcat1 = jnp.concatenate([W1_1, W2_1, W3_1], axis=1)
    bcat1 = jnp.concatenate(
        [b1_1, jnp.zeros((DIM,), jnp.float32), b3_1]).reshape(1, 3 * DIM)
    wcat2 = jnp.concatenate([W1_2, W2_2, W3_2], axis=1)
    bcat2 = jnp.concatenate(
        [b1_2, jnp.zeros((DIM,), jnp.float32), b3_2]).reshape(1, 3 * DIM)

    a1, bb1, c1 = _mm3(x, wcat1, bcat1)
    s1 = _sc_spmm(a1.reshape(NC * N, HD), bb1.reshape(NC * N, HD),
                  src, dst, ew)
    a2, bb2, c2 = _comb_mm(s1, c1, wcat2, bcat2)
    s2 = _sc_spmm(a2.reshape(NC * N, HD), bb2.reshape(NC * N, HD),
                  src, dst, ew)
    return _final(s2, c2)


# speedup vs baseline: 5.2952x; 2.0729x over previous
import functools

import jax
import jax.numpy as jnp
from jax import lax
from jax.experimental import pallas as pl
from jax.experimental.pallas import tpu as pltpu
from jax.experimental.pallas import tpu_sc as plsc

N = 10000
E = 160000
DIM = 256
HD = 128
NC = 2
NS = 16
L = 16
EPT = E // NS
C = 80
NCHUNK = EPT // C
SB = 25
NROWCH = N // C
RCPT = -(-NROWCH // NS)


_BN = 1000


def _mm3_body(x_ref, w_ref, b_ref, a_ref, bb_ref, c_ref):
    acc = jnp.dot(x_ref[...], w_ref[...],
                  preferred_element_type=jnp.float32,
                  precision=lax.Precision.HIGHEST)
    acc = acc + b_ref[...]
    a_ref[...] = acc[:, 0:DIM]
    bb_ref[...] = acc[:, DIM:2 * DIM]
    c_ref[...] = acc[:, 2 * DIM:3 * DIM]


def _mm3(x, wcat, bcat):
    return pl.pallas_call(
        _mm3_body,
        grid=(N // _BN,),
        in_specs=[
            pl.BlockSpec((_BN, DIM), lambda i: (i, 0)),
            pl.BlockSpec((DIM, 3 * DIM), lambda i: (0, 0)),
            pl.BlockSpec((1, 3 * DIM), lambda i: (0, 0)),
        ],
        out_specs=[
            pl.BlockSpec((_BN, DIM), lambda i: (i, 0)),
            pl.BlockSpec((_BN, DIM), lambda i: (i, 0)),
            pl.BlockSpec((_BN, DIM), lambda i: (i, 0)),
        ],
        out_shape=[jax.ShapeDtypeStruct((N, DIM), jnp.float32)] * 3,
    )(x, wcat, bcat)


def _combine(s_ref, bb_ref, c_ref, deg_ref):
    deg = deg_ref[:, 0:1]
    left = s_ref[0] - bb_ref[:, 0:HD] * deg + c_ref[:, 0:HD]
    right = s_ref[1] - bb_ref[:, HD:DIM] * deg + c_ref[:, HD:DIM]
    return jnp.concatenate([left, right], axis=1)


def _comb_mm_body(s_ref, bb_ref, c_ref, deg_ref, w_ref, b_ref,
                  a_ref, bb2_ref, c2_ref):
    h = jnp.maximum(_combine(s_ref, bb_ref, c_ref, deg_ref), 0.0)
    acc = jnp.dot(h, w_ref[...],
                  preferred_element_type=jnp.float32,
                  precision=lax.Precision.HIGHEST)
    acc = acc + b_ref[...]
    a_ref[...] = acc[:, 0:DIM]
    bb2_ref[...] = acc[:, DIM:2 * DIM]
    c2_ref[...] = acc[:, 2 * DIM:3 * DIM]


def _comb_mm(s, bb, c, deg, wcat, bcat):
    return pl.pallas_call(
        _comb_mm_body,
        grid=(N // _BN,),
        in_specs=[
            pl.BlockSpec((NC, _BN, HD), lambda i: (0, i, 0)),
            pl.BlockSpec((_BN, DIM), lambda i: (i, 0)),
            pl.BlockSpec((_BN, DIM), lambda i: (i, 0)),
            pl.BlockSpec((_BN, HD), lambda i: (i, 0)),
            pl.BlockSpec((DIM, 3 * DIM), lambda i: (0, 0)),
            pl.BlockSpec((1, 3 * DIM), lambda i: (0, 0)),
        ],
        out_specs=[
            pl.BlockSpec((_BN, DIM), lambda i: (i, 0)),
            pl.BlockSpec((_BN, DIM), lambda i: (i, 0)),
            pl.BlockSpec((_BN, DIM), lambda i: (i, 0)),
        ],
        out_shape=[jax.ShapeDtypeStruct((N, DIM), jnp.float32)] * 3,
    )(s, bb, c, deg, wcat, bcat)


def _final_body(s_ref, bb_ref, c_ref, deg_ref, out_ref):
    out_ref[...] = _combine(s_ref, bb_ref, c_ref, deg_ref)


def _final(s, bb, c, deg):
    return pl.pallas_call(
        _final_body,
        grid=(N // _BN,),
        in_specs=[
            pl.BlockSpec((NC, _BN, HD), lambda i: (0, i, 0)),
            pl.BlockSpec((_BN, DIM), lambda i: (i, 0)),
            pl.BlockSpec((_BN, DIM), lambda i: (i, 0)),
            pl.BlockSpec((_BN, HD), lambda i: (i, 0)),
        ],
        out_specs=pl.BlockSpec((_BN, DIM), lambda i: (i, 0)),
        out_shape=jax.ShapeDtypeStruct((N, DIM), jnp.float32),
    )(s, bb, c, deg)


_SC_MESH = plsc.VectorSubcoreMesh(
    core_axis_name="c", subcore_axis_name="s", num_cores=NC, num_subcores=NS)


@functools.partial(
    pl.kernel,
    out_type=jax.ShapeDtypeStruct((NC, N, HD), jnp.float32),
    mesh=_SC_MESH,
    scratch_types=[
        pltpu.VMEM((SB * C,), jnp.int32),
        pltpu.VMEM((SB * C,), jnp.int32),
        pltpu.VMEM((SB * C,), jnp.float32),
        [pltpu.VMEM((C,), jnp.int32)] * 2,
        [pltpu.VMEM((C,), jnp.int32)] * 2,
        [pltpu.VMEM((C,), jnp.float32)] * 2,
        [pltpu.VMEM((C, HD), jnp.float32)] * 2,
        pltpu.VMEM_SHARED((N, HD), jnp.float32),
        [pltpu.SemaphoreType.DMA] * 2,
    ],
)
def _sc_spmm(a_hbm, src_hbm, dst_hbm, ew_hbm, s_out,
             src_b, dst_b, ew_b, gidx_cs, dst_cs, ew_cs, rows, s_sh, gsems):
    cid = lax.axis_index("c")
    sid = lax.axis_index("s")
    ebase = sid * EPT

    zero = jnp.zeros((L,), jnp.float32)

    def _zrows_body(e, carry):
        for k in range(HD // L):
            rows[0][e, pl.ds(k * L, L)] = zero
        return carry
    lax.fori_loop(0, C, _zrows_body, 0)

    for j in range(RCPT):
        rch = sid + j * NS

        @pl.when(rch < NROWCH)
        def _zero():
            pltpu.sync_copy(rows[0], s_sh.at[pl.ds(rch * C, C)])
    plsc.subcore_barrier()


    def _prep_and_fire(ch, slot):
        @pl.when(lax.rem(ch, SB) == 0)
        def _stage():
            boff = ebase + ch * C
            pltpu.sync_copy(src_hbm.at[pl.ds(boff, SB * C)], src_b)
            pltpu.sync_copy(dst_hbm.at[pl.ds(boff, SB * C)], dst_b)
            pltpu.sync_copy(ew_hbm.at[pl.ds(boff, SB * C)], ew_b)
        co = pl.multiple_of(lax.rem(ch, SB) * C, C)
        for k in range(C // L):
            sv = src_b[pl.ds(co + k * L, L)]
            gidx_cs[slot][pl.ds(k * L, L)] = sv * 2 + cid
            dst_cs[slot][pl.ds(k * L, L)] = dst_b[pl.ds(co + k * L, L)]
            ew_cs[slot][pl.ds(k * L, L)] = ew_b[pl.ds(co + k * L, L)]
        return pltpu.async_copy(a_hbm.at[gidx_cs[slot]], rows[slot],
                                gsems[slot])

    def _scale_and_scatter(slot):
        for g in range(C // L):
            wv = ew_cs[slot][pl.ds(g * L, L)]
            for lane in range(L):
                w = jnp.full((L,), wv[lane])
                e = g * L + lane
                for k in range(HD // L):
                    rows[slot][e, pl.ds(k * L, L)] = (
                        rows[slot][e, pl.ds(k * L, L)] * w)
        pltpu.sync_copy(rows[slot], s_sh.at[dst_cs[slot]], add=True)

    def _wait_gather(slot):
        pltpu.make_async_copy(a_hbm.at[gidx_cs[slot]], rows[slot],
                              gsems[slot]).wait()

    _prep_and_fire(0, 0)

    def _pair_body(p, carry):
        ch0 = 2 * p
        _prep_and_fire(ch0 + 1, 1)
        _wait_gather(0)
        _scale_and_scatter(0)
        _prep_and_fire(ch0 + 2, 0)
        _wait_gather(1)
        _scale_and_scatter(1)
        return carry
    lax.fori_loop(0, (NCHUNK - 1) // 2, _pair_body, 0)
    _wait_gather(0)
    _scale_and_scatter(0)
    plsc.subcore_barrier()

    for j in range(RCPT):
        rch = sid + j * NS

        @pl.when(rch < NROWCH)
        def _wb():
            pltpu.sync_copy(s_sh.at[pl.ds(rch * C, C)], rows[0])
            pltpu.sync_copy(rows[0], s_out.at[cid, pl.ds(rch * C, C)])


@functools.partial(
    pl.kernel,
    out_type=jax.ShapeDtypeStruct((N, HD), jnp.float32),
    mesh=_SC_MESH,
    scratch_types=[
        pltpu.VMEM((SB * C,), jnp.int32),
        pltpu.VMEM((SB * C,), jnp.float32),
        pltpu.VMEM((C,), jnp.int32),
        pltpu.VMEM((C, HD), jnp.float32),
        pltpu.VMEM_SHARED((N, HD), jnp.float32),
    ],
)
def _sc_deg(dst_hbm, ew_hbm, deg_out, dst_b, ew_b, dst_c, degrow, deg_sh):
    cid = lax.axis_index("c")
    sid = lax.axis_index("s")
    ebase = sid * EPT

    zero = jnp.zeros((L,), jnp.float32)

    def _zrows_body(e, carry):
        for k in range(HD // L):
            degrow[e, pl.ds(k * L, L)] = zero
        return carry
    lax.fori_loop(0, C, _zrows_body, 0)

    for j in range(RCPT):
        rch = sid + j * NS

        @pl.when(rch < NROWCH)
        def _zero():
            pltpu.sync_copy(degrow, deg_sh.at[pl.ds(rch * C, C)])
    plsc.subcore_barrier()

    def _batch_body(b, carry):
        boff = ebase + b * SB * C
        pltpu.sync_copy(dst_hbm.at[pl.ds(boff, SB * C)], dst_b)
        pltpu.sync_copy(ew_hbm.at[pl.ds(boff, SB * C)], ew_b)

        def _chunk_body(j, carry2):
            co = pl.multiple_of(j * C, C)
            for k in range(C // L):
                dst_c[pl.ds(k * L, L)] = dst_b[pl.ds(co + k * L, L)]
            for g in range(C // L):
                wv = ew_b[pl.ds(co + g * L, L)]
                for lane in range(L):
                    w = jnp.full((L,), wv[lane])
                    e = g * L + lane
                    for k in range(HD // L):
                        degrow[e, pl.ds(k * L, L)] = w
            pltpu.sync_copy(degrow, deg_sh.at[dst_c], add=True)
            return carry2
        lax.fori_loop(0, SB, _chunk_body, 0)
        return carry
    lax.fori_loop(0, NCHUNK // SB, _batch_body, 0)
    plsc.subcore_barrier()

    @pl.when(cid == 0)
    def _wb_all():
        for j in range(RCPT):
            rch = sid + j * NS

            @pl.when(rch < NROWCH)
            def _wb():
                pltpu.sync_copy(deg_sh.at[pl.ds(rch * C, C)], degrow)
                pltpu.sync_copy(degrow, deg_out.at[pl.ds(rch * C, C)])


def kernel(x, edge_index, edge_attr, batch,
           W1_1, b1_1, W2_1, W3_1, b3_1,
           W1_2, b1_2, W2_2, W3_2, b3_2):
    del batch
    src = edge_index[0]
    dst = edge_index[1]
    ew = edge_attr.reshape(E)

    wcat1 = jnp.concatenate([W1_1, W2_1, W3_1], axis=1)
    bcat1 = jnp.concatenate(
        [b1_1, jnp.zeros((DIM,), jnp.float32), b3_1]).reshape(1, 3 * DIM)
    wcat2 = jnp.concatenate([W1_2, W2_2, W3_2], axis=1)
    bcat2 = jnp.concatenate(
        [b1_2, jnp.zeros((DIM,), jnp.float32), b3_2]).reshape(1, 3 * DIM)

    deg = _sc_deg(dst, ew)
    a1, bb1, c1 = _mm3(x, wcat1, bcat1)
    s1 = _sc_spmm(a1.reshape(NC * N, HD), src, dst, ew)
    a2, bb2, c2 = _comb_mm(s1, bb1, c1, deg, wcat2, bcat2)
    s2 = _sc_spmm(a2.reshape(NC * N, HD), src, dst, ew)
    return _final(s2, bb2, c2, deg)


# speedup vs baseline: 6.6908x; 1.2636x over previous
import functools

import jax
import jax.numpy as jnp
from jax import lax
from jax.experimental import pallas as pl
from jax.experimental.pallas import tpu as pltpu
from jax.experimental.pallas import tpu_sc as plsc

N = 10000
E = 160000
DIM = 256
HD = 128
NC = 2
NS = 16
L = 16
EPT = E // NS
C = 80
NCHUNK = EPT // C
SB = 25
NROWCH = N // C
RCPT = -(-NROWCH // NS)


_BN = 1000


def _mm3_body(x_ref, w_ref, b_ref, a_ref, bb_ref, c_ref):
    acc = jnp.dot(x_ref[...], w_ref[...],
                  preferred_element_type=jnp.float32,
                  precision=lax.Precision.HIGHEST)
    acc = acc + b_ref[...]
    a_ref[...] = acc[:, 0:DIM]
    bb_ref[...] = acc[:, DIM:2 * DIM]
    c_ref[...] = acc[:, 2 * DIM:3 * DIM]


def _mm3(x, wcat, bcat):
    return pl.pallas_call(
        _mm3_body,
        grid=(N // _BN,),
        in_specs=[
            pl.BlockSpec((_BN, DIM), lambda i: (i, 0)),
            pl.BlockSpec((DIM, 3 * DIM), lambda i: (0, 0)),
            pl.BlockSpec((1, 3 * DIM), lambda i: (0, 0)),
        ],
        out_specs=[
            pl.BlockSpec((_BN, DIM), lambda i: (i, 0)),
            pl.BlockSpec((_BN, DIM), lambda i: (i, 0)),
            pl.BlockSpec((_BN, DIM), lambda i: (i, 0)),
        ],
        out_shape=[jax.ShapeDtypeStruct((N, DIM), jnp.float32)] * 3,
    )(x, wcat, bcat)


def _combine(s_ref, bb_ref, c_ref, deg_ref):
    deg = deg_ref[0, :, 0:1] + deg_ref[1, :, 0:1]
    left = s_ref[0] - bb_ref[:, 0:HD] * deg + c_ref[:, 0:HD]
    right = s_ref[1] - bb_ref[:, HD:DIM] * deg + c_ref[:, HD:DIM]
    return jnp.concatenate([left, right], axis=1)


def _comb_mm_body(s_ref, bb_ref, c_ref, deg_ref, w_ref, b_ref,
                  a_ref, bb2_ref, c2_ref):
    h = jnp.maximum(_combine(s_ref, bb_ref, c_ref, deg_ref), 0.0)
    acc = jnp.dot(h, w_ref[...],
                  preferred_element_type=jnp.float32,
                  precision=lax.Precision.HIGHEST)
    acc = acc + b_ref[...]
    a_ref[...] = acc[:, 0:DIM]
    bb2_ref[...] = acc[:, DIM:2 * DIM]
    c2_ref[...] = acc[:, 2 * DIM:3 * DIM]


def _comb_mm(s, bb, c, deg, wcat, bcat):
    return pl.pallas_call(
        _comb_mm_body,
        grid=(N // _BN,),
        in_specs=[
            pl.BlockSpec((NC, _BN, HD), lambda i: (0, i, 0)),
            pl.BlockSpec((_BN, DIM), lambda i: (i, 0)),
            pl.BlockSpec((_BN, DIM), lambda i: (i, 0)),
            pl.BlockSpec((NC, _BN, _DW), lambda i: (0, i, 0)),
            pl.BlockSpec((DIM, 3 * DIM), lambda i: (0, 0)),
            pl.BlockSpec((1, 3 * DIM), lambda i: (0, 0)),
        ],
        out_specs=[
            pl.BlockSpec((_BN, DIM), lambda i: (i, 0)),
            pl.BlockSpec((_BN, DIM), lambda i: (i, 0)),
            pl.BlockSpec((_BN, DIM), lambda i: (i, 0)),
        ],
        out_shape=[jax.ShapeDtypeStruct((N, DIM), jnp.float32)] * 3,
    )(s, bb, c, deg, wcat, bcat)


def _final_body(s_ref, bb_ref, c_ref, deg_ref, out_ref):
    out_ref[...] = _combine(s_ref, bb_ref, c_ref, deg_ref)


def _final(s, bb, c, deg):
    return pl.pallas_call(
        _final_body,
        grid=(N // _BN,),
        in_specs=[
            pl.BlockSpec((NC, _BN, HD), lambda i: (0, i, 0)),
            pl.BlockSpec((_BN, DIM), lambda i: (i, 0)),
            pl.BlockSpec((_BN, DIM), lambda i: (i, 0)),
            pl.BlockSpec((NC, _BN, _DW), lambda i: (0, i, 0)),
        ],
        out_specs=pl.BlockSpec((_BN, DIM), lambda i: (i, 0)),
        out_shape=jax.ShapeDtypeStruct((N, DIM), jnp.float32),
    )(s, bb, c, deg)


_SC_MESH = plsc.VectorSubcoreMesh(
    core_axis_name="c", subcore_axis_name="s", num_cores=NC, num_subcores=NS)


@functools.partial(
    pl.kernel,
    out_type=jax.ShapeDtypeStruct((NC, N, HD), jnp.float32),
    mesh=_SC_MESH,
    scratch_types=[
        pltpu.VMEM((SB * C,), jnp.int32),
        pltpu.VMEM((SB * C,), jnp.int32),
        pltpu.VMEM((SB * C,), jnp.float32),
        [pltpu.VMEM((C,), jnp.int32)] * 2,
        [pltpu.VMEM((C,), jnp.int32)] * 2,
        [pltpu.VMEM((C,), jnp.float32)] * 2,
        [pltpu.VMEM((C, HD), jnp.float32)] * 2,
        pltpu.VMEM_SHARED((N, HD), jnp.float32),
        [pltpu.SemaphoreType.DMA] * 2,
        [pltpu.SemaphoreType.DMA] * 2,
    ],
)
def _sc_spmm(a_hbm, src_hbm, dst_hbm, ew_hbm, s_out,
             src_b, dst_b, ew_b, gidx_cs, dst_cs, ew_cs, rows, s_sh, gsems,
             ssems):
    cid = lax.axis_index("c")
    sid = lax.axis_index("s")
    ebase = sid * EPT

    zero = jnp.zeros((L,), jnp.float32)

    def _zrows_body(e, carry):
        for k in range(HD // L):
            rows[0][e, pl.ds(k * L, L)] = zero
        return carry
    lax.fori_loop(0, C, _zrows_body, 0)

    for j in range(RCPT):
        rch = sid + j * NS

        @pl.when(rch < NROWCH)
        def _zero():
            pltpu.sync_copy(rows[0], s_sh.at[pl.ds(rch * C, C)])
    plsc.subcore_barrier()


    def _prep_and_fire(ch, slot):
        @pl.when(lax.rem(ch, SB) == 0)
        def _stage():
            boff = ebase + ch * C
            pltpu.sync_copy(src_hbm.at[pl.ds(boff, SB * C)], src_b)
            pltpu.sync_copy(dst_hbm.at[pl.ds(boff, SB * C)], dst_b)
            pltpu.sync_copy(ew_hbm.at[pl.ds(boff, SB * C)], ew_b)
        co = pl.multiple_of(lax.rem(ch, SB) * C, C)
        for k in range(C // L):
            sv = src_b[pl.ds(co + k * L, L)]
            gidx_cs[slot][pl.ds(k * L, L)] = sv * 2 + cid
            dst_cs[slot][pl.ds(k * L, L)] = dst_b[pl.ds(co + k * L, L)]
            ew_cs[slot][pl.ds(k * L, L)] = ew_b[pl.ds(co + k * L, L)]
        return pltpu.async_copy(a_hbm.at[gidx_cs[slot]], rows[slot],
                                gsems[slot])

    def _scale(slot):
        for g in range(C // L):
            wv = ew_cs[slot][pl.ds(g * L, L)]
            for lane in range(L):
                w = jnp.full((L,), wv[lane])
                e = g * L + lane
                for k in range(HD // L):
                    rows[slot][e, pl.ds(k * L, L)] = (
                        rows[slot][e, pl.ds(k * L, L)] * w)

    def _wait_gather(slot):
        pltpu.make_async_copy(a_hbm.at[gidx_cs[slot]], rows[slot],
                              gsems[slot]).wait()

    def _fire_scatter(slot):
        pltpu.sync_copy(rows[slot], s_sh.at[dst_cs[slot]], add=True)

    def _wait_scatter(slot):
        pass

    _prep_and_fire(0, 0)
    _prep_and_fire(1, 1)
    _wait_gather(0)
    _scale(0)
    _fire_scatter(0)

    def _pair_body(p, carry):
        ch = 2 * p + 2
        _wait_scatter(0)
        _prep_and_fire(ch, 0)
        _wait_gather(1)
        _scale(1)
        _fire_scatter(1)
        _wait_scatter(1)
        _prep_and_fire(ch + 1, 1)
        _wait_gather(0)
        _scale(0)
        _fire_scatter(0)
        return carry
    lax.fori_loop(0, (NCHUNK - 3) // 2, _pair_body, 0)
    _wait_scatter(0)
    _prep_and_fire(NCHUNK - 1, 0)
    _wait_gather(1)
    _scale(1)
    _fire_scatter(1)
    _wait_gather(0)
    _scale(0)
    _fire_scatter(0)
    _wait_scatter(1)
    _wait_scatter(0)
    plsc.subcore_barrier()

    for j in range(RCPT):
        rch = sid + j * NS

        @pl.when(rch < NROWCH)
        def _wb():
            pltpu.sync_copy(s_sh.at[pl.ds(rch * C, C)], rows[0])
            pltpu.sync_copy(rows[0], s_out.at[cid, pl.ds(rch * C, C)])


_DW = HD
_NBT = E // (SB * C)
_BPW = -(-_NBT // (NC * NS))


@functools.partial(
    pl.kernel,
    out_type=jax.ShapeDtypeStruct((NC, N, _DW), jnp.float32),
    mesh=_SC_MESH,
    scratch_types=[
        pltpu.VMEM((SB * C,), jnp.int32),
        pltpu.VMEM((SB * C,), jnp.float32),
        pltpu.VMEM((C,), jnp.int32),
        pltpu.VMEM((C, _DW), jnp.float32),
        pltpu.VMEM_SHARED((N, _DW), jnp.float32),
    ],
)
def _sc_deg(dst_hbm, ew_hbm, deg_out, dst_b, ew_b, dst_c, degrow, deg_sh):
    cid = lax.axis_index("c")
    sid = lax.axis_index("s")
    wid = cid * NS + sid

    zero = jnp.zeros((L,), jnp.float32)

    def _zrows_body(e, carry):
        for k in range(_DW // L):
            degrow[e, pl.ds(k * L, L)] = zero
        return carry
    lax.fori_loop(0, C, _zrows_body, 0)

    for j in range(RCPT):
        rch = sid + j * NS

        @pl.when(rch < NROWCH)
        def _zero():
            pltpu.sync_copy(degrow, deg_sh.at[pl.ds(rch * C, C)])
    plsc.subcore_barrier()

    for jb in range(_BPW):
        bi = wid + jb * NC * NS

        @pl.when(bi < _NBT)
        def _batch():
            boff = bi * SB * C
            pltpu.sync_copy(dst_hbm.at[pl.ds(boff, SB * C)], dst_b)
            pltpu.sync_copy(ew_hbm.at[pl.ds(boff, SB * C)], ew_b)

            def _chunk_body(j, carry2):
                co = pl.multiple_of(j * C, C)
                for k in range(C // L):
                    dst_c[pl.ds(k * L, L)] = dst_b[pl.ds(co + k * L, L)]
                for g in range(C // L):
                    wv = ew_b[pl.ds(co + g * L, L)]
                    for lane in range(L):
                        w = jnp.full((L,), wv[lane])
                        e = g * L + lane
                        for k in range(_DW // L):
                            degrow[e, pl.ds(k * L, L)] = w
                pltpu.sync_copy(degrow, deg_sh.at[dst_c], add=True)
                return carry2
            lax.fori_loop(0, SB, _chunk_body, 0)
    plsc.subcore_barrier()

    for j in range(RCPT):
        rch = sid + j * NS

        @pl.when(rch < NROWCH)
        def _wb():
            pltpu.sync_copy(deg_sh.at[pl.ds(rch * C, C)], degrow)
            pltpu.sync_copy(degrow, deg_out.at[cid, pl.ds(rch * C, C)])


def kernel(x, edge_index, edge_attr, batch,
           W1_1, b1_1, W2_1, W3_1, b3_1,
           W1_2, b1_2, W2_2, W3_2, b3_2):
    del batch
    src = edge_index[0]
    dst = edge_index[1]
    ew = edge_attr.reshape(E)

    wcat1 = jnp.concatenate([W1_1, W2_1, W3_1], axis=1)
    bcat1 = jnp.concatenate(
        [b1_1, jnp.zeros((DIM,), jnp.float32), b3_1]).reshape(1, 3 * DIM)
    wcat2 = jnp.concatenate([W1_2, W2_2, W3_2], axis=1)
    bcat2 = jnp.concatenate(
        [b1_2, jnp.zeros((DIM,), jnp.float32), b3_2]).reshape(1, 3 * DIM)

    deg = _sc_deg(dst, ew)
    a1, bb1, c1 = _mm3(x, wcat1, bcat1)
    s1 = _sc_spmm(a1.reshape(NC * N, HD), src, dst, ew)
    a2, bb2, c2 = _comb_mm(s1, bb1, c1, deg, wcat2, bcat2)
    s2 = _sc_spmm(a2.reshape(NC * N, HD), src, dst, ew)
    return _final(s2, bb2, c2, deg)
